# baseline (speedup 1.0000x reference)
import functools

import jax
import jax.numpy as jnp
from jax import lax
from jax.experimental import pallas as pl
from jax.experimental.pallas import tpu as pltpu
from jax.experimental.pallas import tpu_sc as plsc

T = 2048
D = 1024
F = 1024
E = 8
ALPHA = 1.702
BETA = 1.0
LIMIT = 7.0

TM = 256
NB = (2 * T) // TM + E
NPAD = NB * TM

NC = 2
NS = 16
NW = NC * NS
TPW = T // NW
SUB = 16
NCH = TPW // SUB

_MESH = dict(core_axis_name="c", subcore_axis_name="s")


def _meta_body(x_ref, gw_ref, gb_ref,
               p1_ref, p2_ref, w1_ref, w2_ref, bidx_ref, nb_ref):
    x = x_ref[...]
    logits = lax.dot_general(
        x, gw_ref[...], (((1,), (1,)), ((), ())),
        preferred_element_type=jnp.float32) + gb_ref[...]
    iota_e = lax.broadcasted_iota(jnp.int32, logits.shape, 1)
    v1 = jnp.max(logits, axis=1, keepdims=True)
    i1 = jnp.argmax(logits, axis=1)[:, None]
    masked = jnp.where(iota_e == i1, -jnp.inf, logits)
    v2 = jnp.max(masked, axis=1, keepdims=True)
    i2 = jnp.argmax(masked, axis=1)[:, None]
    w1 = 1.0 / (1.0 + jnp.exp(v2 - v1))
    w2 = 1.0 - w1

    one1 = iota_e == i1
    one2 = iota_e == i2
    assign = one1.astype(jnp.float32) + one2.astype(jnp.float32)

    r_iota = lax.broadcasted_iota(jnp.int32, (T, T), 0)
    c_iota = lax.broadcasted_iota(jnp.int32, (T, T), 1)
    lstrict = (r_iota > c_iota).astype(jnp.float32)
    rank = lax.dot_general(
        lstrict, assign, (((1,), (0,)), ((), ())),
        preferred_element_type=jnp.float32)

    cnt_i = jnp.sum(assign, axis=0, keepdims=True).astype(jnp.int32)
    blocks = (cnt_i + TM - 1) // TM
    ee_r = lax.broadcasted_iota(jnp.int32, (E, E), 0)
    ee_c = lax.broadcasted_iota(jnp.int32, (E, E), 1)
    mlt = (ee_r < ee_c).astype(jnp.float32)
    cb = lax.dot_general(
        blocks.astype(jnp.float32), mlt, (((1,), (0,)), ((), ())),
        preferred_element_type=jnp.float32).astype(jnp.int32)
    cb_incl = cb + blocks
    nb = jnp.sum(blocks)

    pos = cb * TM + rank.astype(jnp.int32)
    p1 = jnp.sum(jnp.where(one1, pos, 0), axis=1)
    p2 = jnp.sum(jnp.where(one2, pos, 0), axis=1)

    eidx = lax.broadcasted_iota(jnp.int32, (1, E), 1)
    last_e = jnp.max(jnp.where(blocks > 0, eidx, 0))
    bb = lax.broadcasted_iota(jnp.int32, (NB, E), 0)
    raw = jnp.sum((bb >= cb_incl).astype(jnp.int32), axis=1)
    bidx = jnp.minimum(raw, last_e)

    p1_ref[...] = p1[:, None]
    p2_ref[...] = p2[:, None]
    w1_ref[...] = jnp.broadcast_to(w1, (T, 128))
    w2_ref[...] = jnp.broadcast_to(w2, (T, 128))
    bidx_ref[...] = bidx[:, None]
    nb_ref[...] = jnp.full((1, 1), nb, jnp.int32)


def _meta(x, gate_w, gate_b):
    return pl.pallas_call(
        _meta_body,
        out_shape=[
            jax.ShapeDtypeStruct((T, 1), jnp.int32),
            jax.ShapeDtypeStruct((T, 1), jnp.int32),
            jax.ShapeDtypeStruct((T, 128), jnp.float32),
            jax.ShapeDtypeStruct((T, 128), jnp.float32),
            jax.ShapeDtypeStruct((NB, 1), jnp.int32),
            jax.ShapeDtypeStruct((1, 1), jnp.int32),
        ],
    )(x, gate_w, gate_b.reshape(1, E))


@functools.partial(
    pl.kernel,
    out_type=(jax.ShapeDtypeStruct((NPAD, D), jnp.float32),
              jax.ShapeDtypeStruct((NPAD, 128), jnp.float32)),
    mesh=plsc.VectorSubcoreMesh(**_MESH),
    scratch_types=[
        pltpu.VMEM((TPW,), jnp.int32),
        pltpu.VMEM((TPW,), jnp.int32),
        pltpu.VMEM((TPW, D), jnp.float32),
        pltpu.VMEM((TPW, 128), jnp.float32),
        pltpu.VMEM((TPW, 128), jnp.float32),
        pltpu.SemaphoreType.DMA,
    ],
)
def _dispatch(x_hbm, p1_hbm, p2_hbm, w1_hbm, w2_hbm, xs_hbm, ws_hbm,
              idx1_v, idx2_v, rows_v, wa_v, wb_v, sem):
    wid = lax.axis_index("s") * NC + lax.axis_index("c")
    base = wid * TPW
    pltpu.sync_copy(p1_hbm.at[pl.ds(base, TPW)], idx1_v)
    pltpu.sync_copy(p2_hbm.at[pl.ds(base, TPW)], idx2_v)
    pltpu.sync_copy(x_hbm.at[pl.ds(base, TPW)], rows_v)
    pltpu.sync_copy(w1_hbm.at[pl.ds(base, TPW)], wa_v)
    pltpu.sync_copy(w2_hbm.at[pl.ds(base, TPW)], wb_v)
    c1 = pltpu.async_copy(rows_v, xs_hbm.at[idx1_v], sem)
    c2 = pltpu.async_copy(rows_v, xs_hbm.at[idx2_v], sem)
    c3 = pltpu.async_copy(wa_v, ws_hbm.at[idx1_v], sem)
    c4 = pltpu.async_copy(wb_v, ws_hbm.at[idx2_v], sem)
    c1.wait()
    c2.wait()
    c3.wait()
    c4.wait()


def _grouped_body(bidx_ref, nb_ref, xs_ref, ws_ref,
                  wgu_ref, bc_ref, wd_ref, ys_ref):
    i = pl.program_id(0)

    @pl.when(i < nb_ref[0])
    def _():
        x = xs_ref[...]
        h = lax.dot_general(
            x, wgu_ref[0], (((1,), (1,)), ((), ())),
            preferred_element_type=jnp.float32) + bc_ref[0, :, :2 * F]
        x_glu = jnp.minimum(h[:, :F], LIMIT)
        x_lin = jnp.clip(h[:, F:], -LIMIT, LIMIT)
        act = x_glu * jax.nn.sigmoid(ALPHA * x_glu) * (x_lin + BETA)
        y = lax.dot_general(
            act, wd_ref[0], (((1,), (1,)), ((), ())),
            preferred_element_type=jnp.float32) + bc_ref[0, :, 2 * F:]
        rows = pl.ds(i * TM, TM)
        ys_ref[...] = y * ws_ref[rows, :1]


def _grouped(bidx, nb, xs, ws, w_gate_up, b_gate_up, w_down, b_down):
    grid_spec = pltpu.PrefetchScalarGridSpec(
        num_scalar_prefetch=2,
        grid=(NB,),
        in_specs=[
            pl.BlockSpec((TM, D), lambda i, bidx, nb: (i, 0)),
            pl.BlockSpec((NPAD, 128), lambda i, bidx, nb: (0, 0)),
            pl.BlockSpec((1, 2 * F, D), lambda i, bidx, nb: (bidx[i], 0, 0)),
            pl.BlockSpec((1, 1, 3 * F), lambda i, bidx, nb: (bidx[i], 0, 0)),
            pl.BlockSpec((1, D, F), lambda i, bidx, nb: (bidx[i], 0, 0)),
        ],
        out_specs=pl.BlockSpec((TM, D), lambda i, bidx, nb: (i, 0)),
    )
    return pl.pallas_call(
        _grouped_body,
        grid_spec=grid_spec,
        out_shape=jax.ShapeDtypeStruct((NPAD, D), jnp.float32),
        compiler_params=pltpu.CompilerParams(
            dimension_semantics=("arbitrary",),
        ),
    )(bidx, nb, xs, ws, w_gate_up,
      jnp.concatenate([b_gate_up, b_down], axis=1).reshape(E, 1, 3 * F),
      w_down)


@functools.partial(
    pl.kernel,
    out_type=jax.ShapeDtypeStruct((T, D), jnp.float32),
    mesh=plsc.VectorSubcoreMesh(**_MESH),
    scratch_types=[
        [pltpu.VMEM((SUB,), jnp.int32)] * 2,
        [pltpu.VMEM((SUB,), jnp.int32)] * 2,
        [pltpu.VMEM((SUB, D), jnp.float32)] * 2,
        [pltpu.VMEM((SUB, D), jnp.float32)] * 2,
        [pltpu.VMEM((SUB, D), jnp.float32)] * 2,
        [pltpu.SemaphoreType.DMA] * 2,
        [pltpu.SemaphoreType.DMA] * 2,
    ],
)
def _combine(ys_hbm, p1_hbm, p2_hbm, out_hbm, idx1_v, idx2_v, a_v, b_v,
             o_v, sem, wsem):
    wid = lax.axis_index("s") * NC + lax.axis_index("c")

    def _issue(s_):
        k = s_ % 2
        base = wid * TPW + s_ * SUB
        pltpu.sync_copy(p1_hbm.at[pl.ds(base, SUB)], idx1_v[k])
        pltpu.sync_copy(p2_hbm.at[pl.ds(base, SUB)], idx2_v[k])
        g1 = pltpu.async_copy(ys_hbm.at[idx1_v[k]], a_v[k], sem[k])
        g2 = pltpu.async_copy(ys_hbm.at[idx2_v[k]], b_v[k], sem[k])
        return g1, g2

    pend = _issue(0)
    wpend = [None, None]
    for s_ in range(NCH):
        k = s_ % 2
        base = wid * TPW + s_ * SUB
        pend[0].wait()
        pend[1].wait()
        if s_ + 1 < NCH:
            pend = _issue(s_ + 1)
        if wpend[k] is not None:
            wpend[k].wait()
        for r in range(SUB):

            def _row(j, _, r=r, k=k):
                sl0 = pl.ds(j * 64, 16)
                sl1 = pl.ds(j * 64 + 16, 16)
                sl2 = pl.ds(j * 64 + 32, 16)
                sl3 = pl.ds(j * 64 + 48, 16)
                o_v[k][r, sl0] = a_v[k][r, sl0] + b_v[k][r, sl0]
                o_v[k][r, sl1] = a_v[k][r, sl1] + b_v[k][r, sl1]
                o_v[k][r, sl2] = a_v[k][r, sl2] + b_v[k][r, sl2]
                o_v[k][r, sl3] = a_v[k][r, sl3] + b_v[k][r, sl3]
                return 0

            lax.fori_loop(0, D // 64, _row, 0)
        wpend[k] = pltpu.async_copy(o_v[k], out_hbm.at[pl.ds(base, SUB)],
                                    wsem[k])
    for w in wpend:
        if w is not None:
            w.wait()


@jax.jit
def _moe(x, gate_w, gate_b, w_gate_up, b_gate_up, w_down, b_down):
    p1, p2, w1, w2, bidx, nb = _meta(x, gate_w, gate_b)
    p1 = p1.reshape(T)
    p2 = p2.reshape(T)
    xs, ws = _dispatch(x, p1, p2, w1, w2)
    ys = _grouped(bidx.reshape(NB), nb.reshape(1), xs, ws,
                  w_gate_up, b_gate_up, w_down, b_down)
    return _combine(ys, p1, p2)


def kernel(x, gate_w, gate_b, w_gate_up, b_gate_up, w_down, b_down,
           attn_metadata=0):
    return _moe(x, gate_w, gate_b, w_gate_up, b_gate_up, w_down, b_down)


# speedup vs baseline: 1.0060x; 1.0060x over previous
import functools

import jax
import jax.numpy as jnp
from jax import lax
from jax.experimental import pallas as pl
from jax.experimental.pallas import tpu as pltpu
from jax.experimental.pallas import tpu_sc as plsc

T = 2048
D = 1024
F = 1024
E = 8
ALPHA = 1.702
BETA = 1.0
LIMIT = 7.0

TM = 256
NB = (2 * T) // TM + E
NPAD = NB * TM

NC = 2
NS = 16
NW = NC * NS
TPW = T // NW
SUB = 16
NCH = TPW // SUB

_MESH = dict(core_axis_name="c", subcore_axis_name="s")


def _meta_body(x_ref, gw_ref, gb_ref,
               p1_ref, p2_ref, w1_ref, w2_ref, bidx_ref, nb_ref):
    x = x_ref[...]
    logits = lax.dot_general(
        x, gw_ref[...], (((1,), (1,)), ((), ())),
        preferred_element_type=jnp.float32) + gb_ref[...]
    iota_e = lax.broadcasted_iota(jnp.int32, logits.shape, 1)
    v1 = jnp.max(logits, axis=1, keepdims=True)
    i1 = jnp.argmax(logits, axis=1)[:, None]
    masked = jnp.where(iota_e == i1, -jnp.inf, logits)
    v2 = jnp.max(masked, axis=1, keepdims=True)
    i2 = jnp.argmax(masked, axis=1)[:, None]
    w1 = 1.0 / (1.0 + jnp.exp(v2 - v1))
    w2 = 1.0 - w1

    one1 = iota_e == i1
    one2 = iota_e == i2
    assign = one1.astype(jnp.float32) + one2.astype(jnp.float32)

    r_iota = lax.broadcasted_iota(jnp.int32, (T, T), 0)
    c_iota = lax.broadcasted_iota(jnp.int32, (T, T), 1)
    lstrict = (r_iota > c_iota).astype(jnp.float32)
    rank = lax.dot_general(
        lstrict, assign, (((1,), (0,)), ((), ())),
        preferred_element_type=jnp.float32)

    cnt_i = jnp.sum(assign, axis=0, keepdims=True).astype(jnp.int32)
    blocks = (cnt_i + TM - 1) // TM
    ee_r = lax.broadcasted_iota(jnp.int32, (E, E), 0)
    ee_c = lax.broadcasted_iota(jnp.int32, (E, E), 1)
    mlt = (ee_r < ee_c).astype(jnp.float32)
    cb = lax.dot_general(
        blocks.astype(jnp.float32), mlt, (((1,), (0,)), ((), ())),
        preferred_element_type=jnp.float32).astype(jnp.int32)
    cb_incl = cb + blocks
    nb = jnp.sum(blocks)

    pos = cb * TM + rank.astype(jnp.int32)
    p1 = jnp.sum(jnp.where(one1, pos, 0), axis=1)
    p2 = jnp.sum(jnp.where(one2, pos, 0), axis=1)

    eidx = lax.broadcasted_iota(jnp.int32, (1, E), 1)
    last_e = jnp.max(jnp.where(blocks > 0, eidx, 0))
    bb = lax.broadcasted_iota(jnp.int32, (NB, E), 0)
    raw = jnp.sum((bb >= cb_incl).astype(jnp.int32), axis=1)
    bidx = jnp.minimum(raw, last_e)

    p1_ref[...] = p1[:, None]
    p2_ref[...] = p2[:, None]
    w1_ref[...] = jnp.broadcast_to(w1, (T, 128))
    w2_ref[...] = jnp.broadcast_to(w2, (T, 128))
    bidx_ref[...] = bidx[:, None]
    nb_ref[...] = jnp.full((1, 1), nb, jnp.int32)


def _meta(x, gate_w, gate_b):
    return pl.pallas_call(
        _meta_body,
        out_shape=[
            jax.ShapeDtypeStruct((T, 1), jnp.int32),
            jax.ShapeDtypeStruct((T, 1), jnp.int32),
            jax.ShapeDtypeStruct((T, 128), jnp.float32),
            jax.ShapeDtypeStruct((T, 128), jnp.float32),
            jax.ShapeDtypeStruct((NB, 1), jnp.int32),
            jax.ShapeDtypeStruct((1, 1), jnp.int32),
        ],
    )(x, gate_w, gate_b.reshape(1, E))


@functools.partial(
    pl.kernel,
    out_type=(jax.ShapeDtypeStruct((NPAD, D), jnp.float32),
              jax.ShapeDtypeStruct((NPAD, 128), jnp.float32)),
    mesh=plsc.VectorSubcoreMesh(**_MESH),
    scratch_types=[
        pltpu.VMEM((TPW,), jnp.int32),
        pltpu.VMEM((TPW,), jnp.int32),
        pltpu.VMEM((TPW, D), jnp.float32),
        pltpu.VMEM((TPW, 128), jnp.float32),
        pltpu.VMEM((TPW, 128), jnp.float32),
        pltpu.SemaphoreType.DMA,
    ],
)
def _dispatch(x_hbm, p1_hbm, p2_hbm, w1_hbm, w2_hbm, xs_hbm, ws_hbm,
              idx1_v, idx2_v, rows_v, wa_v, wb_v, sem):
    wid = lax.axis_index("s") * NC + lax.axis_index("c")
    base = wid * TPW
    pltpu.sync_copy(p1_hbm.at[pl.ds(base, TPW)], idx1_v)
    pltpu.sync_copy(p2_hbm.at[pl.ds(base, TPW)], idx2_v)
    pltpu.sync_copy(x_hbm.at[pl.ds(base, TPW)], rows_v)
    pltpu.sync_copy(w1_hbm.at[pl.ds(base, TPW)], wa_v)
    pltpu.sync_copy(w2_hbm.at[pl.ds(base, TPW)], wb_v)
    c1 = pltpu.async_copy(rows_v, xs_hbm.at[idx1_v], sem)
    c2 = pltpu.async_copy(rows_v, xs_hbm.at[idx2_v], sem)
    c3 = pltpu.async_copy(wa_v, ws_hbm.at[idx1_v], sem)
    c4 = pltpu.async_copy(wb_v, ws_hbm.at[idx2_v], sem)
    c1.wait()
    c2.wait()
    c3.wait()
    c4.wait()


def _grouped_body(bidx_ref, nb_ref, xs_ref, ws_ref,
                  wgu_ref, bgu_ref, wd_ref, bd_ref, ys_ref):
    i = pl.program_id(0)

    @pl.when(i < nb_ref[0])
    def _():
        x = xs_ref[...]
        h = lax.dot_general(
            x, wgu_ref[0], (((1,), (1,)), ((), ())),
            preferred_element_type=jnp.float32) + bgu_ref[0]
        x_glu = jnp.minimum(h[:, :F], LIMIT)
        x_lin = jnp.clip(h[:, F:], -LIMIT, LIMIT)
        act = x_glu * jax.nn.sigmoid(ALPHA * x_glu) * (x_lin + BETA)
        y = lax.dot_general(
            act, wd_ref[0], (((1,), (1,)), ((), ())),
            preferred_element_type=jnp.float32) + bd_ref[0]
        ys_ref[...] = y * ws_ref[:, :1]


def _grouped(bidx, nb, xs, ws, w_gate_up, b_gate_up, w_down, b_down):
    grid_spec = pltpu.PrefetchScalarGridSpec(
        num_scalar_prefetch=2,
        grid=(NB,),
        in_specs=[
            pl.BlockSpec((TM, D), lambda i, bidx, nb: (i, 0)),
            pl.BlockSpec((TM, 128), lambda i, bidx, nb: (i, 0)),
            pl.BlockSpec((1, 2 * F, D), lambda i, bidx, nb: (bidx[i], 0, 0)),
            pl.BlockSpec((1, 1, 2 * F), lambda i, bidx, nb: (bidx[i], 0, 0)),
            pl.BlockSpec((1, D, F), lambda i, bidx, nb: (bidx[i], 0, 0)),
            pl.BlockSpec((1, 1, D), lambda i, bidx, nb: (bidx[i], 0, 0)),
        ],
        out_specs=pl.BlockSpec((TM, D), lambda i, bidx, nb: (i, 0)),
    )
    return pl.pallas_call(
        _grouped_body,
        grid_spec=grid_spec,
        out_shape=jax.ShapeDtypeStruct((NPAD, D), jnp.float32),
        compiler_params=pltpu.CompilerParams(
            dimension_semantics=("arbitrary",),
        ),
    )(bidx, nb, xs, ws, w_gate_up, b_gate_up.reshape(E, 1, 2 * F),
      w_down, b_down.reshape(E, 1, D))


@functools.partial(
    pl.kernel,
    out_type=jax.ShapeDtypeStruct((T, D), jnp.float32),
    mesh=plsc.VectorSubcoreMesh(**_MESH),
    scratch_types=[
        [pltpu.VMEM((SUB,), jnp.int32)] * 2,
        [pltpu.VMEM((SUB,), jnp.int32)] * 2,
        [pltpu.VMEM((SUB, D), jnp.float32)] * 2,
        [pltpu.VMEM((SUB, D), jnp.float32)] * 2,
        [pltpu.VMEM((SUB, D), jnp.float32)] * 2,
        [pltpu.SemaphoreType.DMA] * 2,
        [pltpu.SemaphoreType.DMA] * 2,
    ],
)
def _combine(ys_hbm, p1_hbm, p2_hbm, out_hbm, idx1_v, idx2_v, a_v, b_v,
             o_v, sem, wsem):
    wid = lax.axis_index("s") * NC + lax.axis_index("c")

    def _issue(s_):
        k = s_ % 2
        base = wid * TPW + s_ * SUB
        pltpu.sync_copy(p1_hbm.at[pl.ds(base, SUB)], idx1_v[k])
        pltpu.sync_copy(p2_hbm.at[pl.ds(base, SUB)], idx2_v[k])
        g1 = pltpu.async_copy(ys_hbm.at[idx1_v[k]], a_v[k], sem[k])
        g2 = pltpu.async_copy(ys_hbm.at[idx2_v[k]], b_v[k], sem[k])
        return g1, g2

    pend = _issue(0)
    wpend = [None, None]
    for s_ in range(NCH):
        k = s_ % 2
        base = wid * TPW + s_ * SUB
        pend[0].wait()
        pend[1].wait()
        if s_ + 1 < NCH:
            pend = _issue(s_ + 1)
        if wpend[k] is not None:
            wpend[k].wait()
        for r in range(SUB):

            def _row(j, _, r=r, k=k):
                sl0 = pl.ds(j * 64, 16)
                sl1 = pl.ds(j * 64 + 16, 16)
                sl2 = pl.ds(j * 64 + 32, 16)
                sl3 = pl.ds(j * 64 + 48, 16)
                o_v[k][r, sl0] = a_v[k][r, sl0] + b_v[k][r, sl0]
                o_v[k][r, sl1] = a_v[k][r, sl1] + b_v[k][r, sl1]
                o_v[k][r, sl2] = a_v[k][r, sl2] + b_v[k][r, sl2]
                o_v[k][r, sl3] = a_v[k][r, sl3] + b_v[k][r, sl3]
                return 0

            lax.fori_loop(0, D // 64, _row, 0)
        wpend[k] = pltpu.async_copy(o_v[k], out_hbm.at[pl.ds(base, SUB)],
                                    wsem[k])
    for w in wpend:
        if w is not None:
            w.wait()


@jax.jit
def _moe(x, gate_w, gate_b, w_gate_up, b_gate_up, w_down, b_down):
    p1, p2, w1, w2, bidx, nb = _meta(x, gate_w, gate_b)
    p1 = p1.reshape(T)
    p2 = p2.reshape(T)
    xs, ws = _dispatch(x, p1, p2, w1, w2)
    ys = _grouped(bidx.reshape(NB), nb.reshape(1), xs, ws,
                  w_gate_up, b_gate_up, w_down, b_down)
    return _combine(ys, p1, p2)


def kernel(x, gate_w, gate_b, w_gate_up, b_gate_up, w_down, b_down,
           attn_metadata=0):
    return _moe(x, gate_w, gate_b, w_gate_up, b_gate_up, w_down, b_down)
